# Initial kernel scaffold; baseline (speedup 1.0000x reference)
import jax
import jax.numpy as jnp
from jax.experimental import pallas as pl


def kernel(features, targets):
    raise NotImplementedError("write your pallas kernel here")


# speedup vs baseline: 1.1660x; 1.1660x over previous
import functools

import jax
import jax.numpy as jnp
from jax import lax
from jax.experimental import pallas as pl
from jax.experimental.pallas import tpu as pltpu
from jax.experimental.pallas import tpu_sc as plsc

_NG = 2
_HW = 64 * 64
_NSPOT = 50
_TROW = 256
_LANES = 16
_CHUNKS = _HW // _LANES
_GROUP = 10
_LN2 = 0.6931471805599453


def _vlog(x):
    xi = lax.bitcast_convert_type(x, jnp.int32)
    e = lax.shift_right_arithmetic(xi, 23) - 127
    m = lax.bitcast_convert_type((xi & 0x007FFFFF) | 0x3F800000, jnp.float32)
    big = m > 1.4142135623730951
    m = jnp.where(big, m * 0.5, m)
    e = jnp.where(big, e + 1, e).astype(jnp.float32)
    t = (m - 1.0) / (m + 1.0)
    t2 = t * t
    p = 2.0 + t2 * (2.0 / 3.0 + t2 * (2.0 / 5.0 + t2 * (2.0 / 7.0 + t2 * (2.0 / 9.0))))
    return e * _LN2 + t * p


def _splat_word(ref, word):
    chunk, lane = divmod(word, _LANES)
    vec = ref[pl.ds(chunk * _LANES, _LANES)]
    return _shuffle(vec, jnp.full((_LANES,), lane, jnp.int32))


_GATHER_DNUMS = lax.GatherDimensionNumbers(
    offset_dims=(), collapsed_slice_dims=(0,), start_index_map=(0,))


def _shuffle(x, idx):
    return lax.gather(x, idx[:, None], _GATHER_DNUMS, (1,),
                      mode=lax.GatherScatterMode.PROMISE_IN_BOUNDS)


def _hreduce(x, op):
    idx = lax.iota(jnp.int32, _LANES)
    for k in (1, 2, 4, 8):
        x = op(x, _shuffle(x, idx ^ k))
    return x


def _gmm_body(feat_hbm, tgt_hbm, out_hbm, feat_v, tgt_v, r1_v, r2_v, c_v, out_v):
    cid = lax.axis_index("c")
    sid = lax.axis_index("s")
    wid = sid * 2 + cid

    pltpu.sync_copy(feat_hbm.at[wid], feat_v)
    pltpu.sync_copy(tgt_hbm.at[wid], tgt_v)

    zero = jnp.zeros((_LANES,), jnp.float32)

    def prep(i, carry):
        mx, sp = carry
        sl = pl.ds(i * _LANES, _LANES)
        p = jnp.maximum(feat_v[pl.ds(0 * _HW + i * _LANES, _LANES)], 1e-20)
        s1 = jnp.maximum(feat_v[pl.ds(3 * _HW + i * _LANES, _LANES)], 1e-10)
        s2 = jnp.maximum(feat_v[pl.ds(4 * _HW + i * _LANES, _LANES)], 1e-10)
        r1_v[sl] = 0.5 / (s1 * s1)
        r2_v[sl] = 0.5 / (s2 * s2)
        c = _vlog(p / (s1 * s2))
        c_v[sl] = c
        return jnp.maximum(mx, c), sp + p

    mx, sp = lax.fori_loop(0, _CHUNKS, prep, (jnp.full((_LANES,), -3.0e38, jnp.float32), zero))
    mhat_v = _hreduce(mx, jnp.maximum)
    kshift_v = mhat_v - _vlog(_hreduce(sp, jnp.add))

    def shift(i, carry):
        sl = pl.ds(i * _LANES, _LANES)
        c_v[sl] = c_v[sl] - mhat_v
        return carry

    lax.fori_loop(0, _CHUNKS, shift, 0)

    loss_v = zero
    msum_v = zero
    for g in range(0, _NSPOT, _GROUP):
        spots = list(range(g, g + _GROUP))
        tv1 = [_splat_word(tgt_v, 3 * s + 1) for s in spots]
        tv2 = [_splat_word(tgt_v, 3 * s + 2) for s in spots]

        def body(i, accs):
            sl = pl.ds(i * _LANES, _LANES)
            c = c_v[sl]
            r1 = r1_v[sl]
            r2 = r2_v[sl]
            m1 = feat_v[pl.ds(1 * _HW + i * _LANES, _LANES)]
            m2 = feat_v[pl.ds(2 * _HW + i * _LANES, _LANES)]
            out = []
            for j in range(_GROUP):
                d1 = tv1[j] - m1
                d2 = tv2[j] - m2
                out.append(accs[j] + jnp.exp(c - r1 * (d1 * d1) - r2 * (d2 * d2)))
            return tuple(out)

        accs = lax.fori_loop(0, _CHUNKS, body, tuple(zero for _ in spots))
        for j, s in enumerate(spots):
            ssum = jnp.maximum(_hreduce(accs[j], jnp.add), 1e-37)
            logv = _vlog(ssum)
            mask = _splat_word(tgt_v, 3 * s + 0)
            loss_v = loss_v + mask * logv
            msum_v = msum_v + mask

    out_v[pl.ds(0, _LANES)] = -(loss_v + msum_v * kshift_v)
    pltpu.sync_copy(out_v, out_hbm.at[wid])


@functools.partial(jax.jit, static_argnums=())
def _gmm_call(feats, tgt):
    bf = feats.shape[0]
    run = pl.kernel(
        _gmm_body,
        out_type=jax.ShapeDtypeStruct((bf, _LANES), jnp.float32),
        mesh=plsc.VectorSubcoreMesh(core_axis_name="c", subcore_axis_name="s"),
        scratch_types=[
            pltpu.VMEM((5 * _HW,), jnp.float32),
            pltpu.VMEM((_TROW,), jnp.float32),
            pltpu.VMEM((_HW,), jnp.float32),
            pltpu.VMEM((_HW,), jnp.float32),
            pltpu.VMEM((_HW,), jnp.float32),
            pltpu.VMEM((_LANES,), jnp.float32),
        ],
    )
    return run(feats, tgt)


def kernel(features, targets):
    B, F, nf, h, w = features.shape
    max_spots = targets.shape[2]
    assert nf == 2 * _NG + 1 and h * w == _HW and max_spots == _NSPOT
    feats = features.reshape(B * F, nf * h * w)
    tgt = targets.reshape(B * F, max_spots * (_NG + 1))
    tgt = jnp.pad(tgt, ((0, 0), (0, _TROW - tgt.shape[1])))
    out = _gmm_call(feats, tgt)
    return out[:, 0].reshape(B, F)
